# Initial kernel scaffold; baseline (speedup 1.0000x reference)
import jax
import jax.numpy as jnp
from jax.experimental import pallas as pl


def kernel(x, edge_index, W1, b1, W2, b2, lin1_W, lin1_b, lin3_W, lin3_b):
    raise NotImplementedError("write your pallas kernel here")


# speedup vs baseline: 1.0023x; 1.0023x over previous
import functools

import jax
import jax.numpy as jnp
from jax.experimental import pallas as pl

N_NODES = 1024
HID = 8
FLAT = N_NODES * HID
MID = FLAT // 2
OUT = 256
NBLK = 16
BLK = MID // NBLK


def _gcn_conv(x, edge_index, W, b):
    n = x.shape[0]
    src = edge_index[0]
    dst = edge_index[1]
    loop = jnp.arange(n, dtype=edge_index.dtype)
    src = jnp.concatenate([src, loop])
    dst = jnp.concatenate([dst, loop])
    deg = jnp.zeros((n,), jnp.float32).at[dst].add(1.0)
    dinv = jax.lax.rsqrt(jnp.maximum(deg, 1e-12))
    norm = dinv[src] * dinv[dst]
    xw = x @ W
    msg = xw[src] * norm[:, None]
    out = jnp.zeros((n, W.shape[1]), jnp.float32).at[dst].add(msg)
    return out + b


def _mlp_body(v_ref, w1_ref, b1_ref, w3_ref, b3_ref, o_ref, acc_ref):
    i = pl.program_id(0)
    s = jnp.dot(v_ref[...], w1_ref[...], preferred_element_type=jnp.float32)
    s = jnp.maximum(s + b1_ref[...], 0.0)
    part = jnp.dot(s, w3_ref[...], preferred_element_type=jnp.float32)

    @pl.when(i == 0)
    def _():
        acc_ref[...] = part

    @pl.when(i > 0)
    def _():
        acc_ref[...] = acc_ref[...] + part

    @pl.when(i == NBLK - 1)
    def _():
        t = acc_ref[...] + b3_ref[...]
        m = jnp.max(t, axis=-1, keepdims=True)
        e = jnp.exp(t - m)
        o_ref[...] = e / jnp.sum(e, axis=-1, keepdims=True)


def _mlp_head(v, lin1_W, lin1_b, lin3_W, lin3_b):
    v2 = v.reshape(1, FLAT)
    b1 = lin1_b.reshape(1, MID)
    b3 = lin3_b.reshape(1, OUT)
    out = pl.pallas_call(
        _mlp_body,
        grid=(NBLK,),
        in_specs=[
            pl.BlockSpec((1, FLAT), lambda i: (0, 0)),
            pl.BlockSpec((FLAT, BLK), lambda i: (0, i)),
            pl.BlockSpec((1, BLK), lambda i: (0, i)),
            pl.BlockSpec((BLK, OUT), lambda i: (i, 0)),
            pl.BlockSpec((1, OUT), lambda i: (0, 0)),
        ],
        out_specs=pl.BlockSpec((1, OUT), lambda i: (0, 0)),
        out_shape=jax.ShapeDtypeStruct((1, OUT), jnp.float32),
        scratch_shapes=[pltpu_vmem((1, OUT), jnp.float32)],
    )(v2, lin1_W, b1, lin3_W, b3)
    return out.reshape(OUT)


def pltpu_vmem(shape, dtype):
    from jax.experimental.pallas import tpu as pltpu
    return pltpu.VMEM(shape, dtype)


def kernel(x, edge_index, W1, b1, W2, b2, lin1_W, lin1_b, lin3_W, lin3_b):
    h = jax.nn.relu(_gcn_conv(x, edge_index, W1, b1))
    h = _gcn_conv(h, edge_index, W2, b2)
    v = h.reshape(-1)
    return _mlp_head(v, lin1_W, lin1_b, lin3_W, lin3_b)
